# baseline (speedup 1.0000x reference)
import functools

import jax
import jax.numpy as jnp
from jax import lax
from jax.experimental import pallas as pl
from jax.experimental.pallas import tpu as pltpu
from jax.experimental.pallas import tpu_sc as plsc

_N = 10000
_E = 320000
_NP = 10240
_EP = 327680
_PAD = _EP - _E
_DUMP = _N
_CHUNK = 128
_NTILES = 16
_RPT = _NP // _NTILES

@functools.cache
def _get_sc_deg(width=128):
    mesh = plsc.VectorSubcoreMesh(core_axis_name="c", subcore_axis_name="s")

    @functools.partial(
        pl.kernel,
        mesh=mesh,
        out_type=jax.ShapeDtypeStruct((2 * _NP, width), jnp.float32),
        scratch_types=[
            pltpu.VMEM((_EP // _CHUNK // 2 // _NTILES, 128), jnp.int32),
            pltpu.VMEM((_CHUNK, width), jnp.float32),
            pltpu.VMEM_SHARED((_NP, width), jnp.float32),
            pltpu.SemaphoreType.DMA((4,)),
        ],
    )
    def _sc_deg(dst_hbm, ones_hbm, zeros_hbm, out_hbm, dstall, onesbuf, acc,
                ssem):
        cid = lax.axis_index("c")
        tid = lax.axis_index("s")
        per_core = _EP // _CHUNK // 2
        per_tile = per_core // _NTILES
        base = cid * per_core + tid * per_tile
        pltpu.sync_copy(dst_hbm.at[pl.ds(base, per_tile)], dstall)
        pltpu.sync_copy(ones_hbm, onesbuf)
        for k in range(_RPT // _CHUNK):
            pltpu.sync_copy(zeros_hbm,
                            acc.at[pl.ds(tid * _RPT + k * _CHUNK, _CHUNK)])
        plsc.subcore_barrier()

        def body(i, carry):
            s = lax.rem(i, 4)

            @pl.when(i >= 4)
            def _wait():
                pltpu.make_async_copy(onesbuf, acc.at[dstall.at[i - 4]],
                                      ssem.at[s]).wait()

            pltpu.async_copy(onesbuf, acc.at[dstall.at[i]], ssem.at[s],
                             add=True)
            return carry

        lax.fori_loop(0, per_tile, body, 0)
        for k in range(4):
            s = (per_tile - 4 + k) % 4
            pltpu.make_async_copy(onesbuf, acc.at[dstall.at[per_tile - 4 + k]],
                                  ssem.at[s]).wait()
        plsc.subcore_barrier()
        pltpu.sync_copy(acc.at[pl.ds(tid * _RPT, _RPT)],
                        out_hbm.at[pl.ds(cid * _NP + tid * _RPT, _RPT)])

    return _sc_deg


@functools.cache
def _make_sc_pass(per_core_chunks, split_edges, src_bias):
    per_tile = per_core_chunks // _NTILES
    iblk = 40
    nblocks = per_tile // iblk
    mesh = plsc.VectorSubcoreMesh(core_axis_name="c", subcore_axis_name="s")

    @functools.partial(
        pl.kernel,
        mesh=mesh,
        out_type=jax.ShapeDtypeStruct((2 * _NP, 128), jnp.float32),
        scratch_types=[
            pltpu.VMEM((iblk, 128), jnp.int32),
            pltpu.VMEM((iblk, 128), jnp.int32),
            pltpu.VMEM((2, _CHUNK, 128), jnp.float32),
            pltpu.VMEM_SHARED((_NP, 128), jnp.float32),
            pltpu.SemaphoreType.DMA((2,)),
            pltpu.SemaphoreType.DMA((2,)),
        ],
    )
    def _sc_pass(table_hbm, src_hbm, dst_hbm, zeros_hbm, out_hbm,
                 srcall, dstall, gbuf, acc, gsem, ssem):
        cid = lax.axis_index("c")
        tid = lax.axis_index("s")
        base = tid * per_tile
        if split_edges:
            base = base + cid * per_core_chunks
        pltpu.sync_copy(zeros_hbm, gbuf.at[0])
        for k in range(_RPT // _CHUNK):
            pltpu.sync_copy(gbuf.at[0],
                            acc.at[pl.ds(tid * _RPT + k * _CHUNK, _CHUNK)])
        plsc.subcore_barrier()

        def outer(bi, carry):
            blk = base + bi * iblk
            pltpu.sync_copy(src_hbm.at[pl.ds(blk, iblk)], srcall)
            pltpu.sync_copy(dst_hbm.at[pl.ds(blk, iblk)], dstall)
            if src_bias:
                @pl.when(cid == 1)
                def _bias():
                    bvec = jnp.full((16,), src_bias, jnp.int32)

                    def bias_body(j, c3):
                        for k in range(_CHUNK // 16):
                            srcall[j, pl.ds(k * 16, 16)] = (
                                srcall[j, pl.ds(k * 16, 16)] + bvec)
                        return c3

                    lax.fori_loop(0, iblk, bias_body, 0)
            pltpu.async_copy(table_hbm.at[srcall.at[0]], gbuf.at[0], gsem.at[0])
            for i in range(iblk):
                b = i % 2
                nb = 1 - b
                if i + 1 < iblk:
                    if i >= 1:
                        pltpu.make_async_copy(gbuf.at[nb],
                                              acc.at[dstall.at[i - 1]],
                                              ssem.at[nb]).wait()
                    pltpu.async_copy(table_hbm.at[srcall.at[i + 1]],
                                     gbuf.at[nb], gsem.at[nb])
                pltpu.make_async_copy(table_hbm.at[srcall.at[i]], gbuf.at[b],
                                      gsem.at[b]).wait()
                pltpu.async_copy(gbuf.at[b], acc.at[dstall.at[i]], ssem.at[b],
                                 add=True)
            for i in (iblk - 2, iblk - 1):
                pltpu.make_async_copy(gbuf.at[i % 2], acc.at[dstall.at[i]],
                                      ssem.at[i % 2]).wait()
            return carry

        lax.fori_loop(0, nblocks, outer, 0)
        plsc.subcore_barrier()
        pltpu.sync_copy(acc.at[pl.ds(tid * _RPT, _RPT)],
                        out_hbm.at[pl.ds(cid * _NP + tid * _RPT, _RPT)])

    return _sc_pass


_ROWS = 512
_NB = _NP // _ROWS

_DN = (((1,), (0,)), ((), ()))


def _dinv_of(d0, d1):
    return lax.rsqrt(d0[:, 0:1] + d1[:, 0:1] + 1.0)


def _tc1_body(x_ref, w1_ref, d0_ref, d1_ref, o_ref):
    dinv = _dinv_of(d0_ref, d1_ref)
    acc = lax.dot_general(x_ref[...], w1_ref[...], _DN,
                          precision=lax.Precision.DEFAULT,
                          preferred_element_type=jnp.float32)
    o_ref[...] = (acc * dinv)[None]


_tc1 = pl.pallas_call(
    _tc1_body,
    grid=(2, _NB := _NP // 512),
    in_specs=[
        pl.BlockSpec((512, 128), lambda h, i: (i, 0)),
        pl.BlockSpec((128, 128), lambda h, i: (0, h)),
        pl.BlockSpec((512, 128), lambda h, i: (i, 0)),
        pl.BlockSpec((512, 128), lambda h, i: (i, 0)),
    ],
    out_specs=pl.BlockSpec((1, 512, 128), lambda h, i: (h, i, 0)),
    out_shape=jax.ShapeDtypeStruct((2, _NP, 128), jnp.float32),
)


def _tc2_body(a0_ref, a1_ref, p0_ref, p1_ref, d0_ref, d1_ref, b1_ref, w2_ref,
              o_ref):
    dinv = _dinv_of(d0_ref, d1_ref)
    h_a = jnp.maximum(dinv * (a0_ref[...] + p0_ref[...]) + b1_ref[0:1, :], 0.0)
    h_b = jnp.maximum(dinv * (a1_ref[...] + p1_ref[...]) + b1_ref[1:2, :], 0.0)
    acc = lax.dot_general(h_a, w2_ref[0:128, :], _DN,
                          precision=lax.Precision.DEFAULT,
                          preferred_element_type=jnp.float32)
    acc += lax.dot_general(h_b, w2_ref[128:256, :], _DN,
                           precision=lax.Precision.DEFAULT,
                           preferred_element_type=jnp.float32)
    o_ref[...] = acc * dinv


_tc2 = pl.pallas_call(
    _tc2_body,
    grid=(_NB,),
    in_specs=[
        pl.BlockSpec((512, 128), lambda i: (i, 0)),
        pl.BlockSpec((512, 128), lambda i: (i, 0)),
        pl.BlockSpec((512, 128), lambda i: (i, 0)),
        pl.BlockSpec((512, 128), lambda i: (i, 0)),
        pl.BlockSpec((512, 128), lambda i: (i, 0)),
        pl.BlockSpec((512, 128), lambda i: (i, 0)),
        pl.BlockSpec((2, 128), lambda i: (0, 0)),
        pl.BlockSpec((256, 128), lambda i: (0, 0)),
    ],
    out_specs=pl.BlockSpec((512, 128), lambda i: (i, 0)),
    out_shape=jax.ShapeDtypeStruct((_NP, 128), jnp.float32),
)


def _tc3_body(a0_ref, a1_ref, p2_ref, d0_ref, d1_ref, b2_ref, o_ref):
    dinv = _dinv_of(d0_ref, d1_ref)
    o_ref[...] = dinv * (a0_ref[...] + a1_ref[...] + p2_ref[...]) + b2_ref[...]


_tc3 = pl.pallas_call(
    _tc3_body,
    grid=(_NB,),
    in_specs=[
        pl.BlockSpec((512, 128), lambda i: (i, 0)),
        pl.BlockSpec((512, 128), lambda i: (i, 0)),
        pl.BlockSpec((512, 128), lambda i: (i, 0)),
        pl.BlockSpec((512, 128), lambda i: (i, 0)),
        pl.BlockSpec((512, 128), lambda i: (i, 0)),
        pl.BlockSpec((1, 128), lambda i: (0, 0)),
    ],
    out_specs=pl.BlockSpec((512, 128), lambda i: (i, 0)),
    out_shape=jax.ShapeDtypeStruct((_NP, 128), jnp.float32),
)


def kernel(x, edge_index, w1, b1, w2, b2):
    f32 = jnp.float32
    src = edge_index[0].astype(jnp.int32)
    dst = edge_index[1].astype(jnp.int32)
    pad_idx = jnp.arange(_PAD, dtype=jnp.int32)
    src_p = jnp.concatenate([src, pad_idx % _N])
    dst_p = jnp.concatenate([dst, _DUMP + pad_idx % (_NP - _N)])
    src_p = src_p.reshape(-1, _CHUNK)
    dst_p = dst_p.reshape(-1, _CHUNK)
    zeros128 = jnp.zeros((_CHUNK, 128), f32)
    ones128 = jnp.ones((_CHUNK, 128), f32)
    xp = jnp.pad(x, ((0, _NP - _N), (0, 0)))

    degpart = _get_sc_deg()(dst_p, ones128, zeros128)
    d0, d1 = degpart[:_NP], degpart[_NP:]
    p1s = _tc1(xp, w1, d0, d1)
    p1flat = p1s.reshape(2 * _NP, 128)
    a1 = _make_sc_pass(_EP // _CHUNK, False, _NP)(p1flat, src_p, dst_p,
                                                  zeros128)
    p2 = _tc2(a1[:_NP], a1[_NP:], p1s[0], p1s[1], d0, d1,
              b1.reshape(2, 128), w2)
    a2 = _make_sc_pass(_EP // _CHUNK // 2, True, 0)(p2, src_p, dst_p, zeros128)
    out = _tc3(a2[:_NP], a2[_NP:], p2, d0, d1, b2.reshape(1, 128))
    return out[:_N]


# speedup vs baseline: 1.0019x; 1.0019x over previous
import functools

import jax
import jax.numpy as jnp
from jax import lax
from jax.experimental import pallas as pl
from jax.experimental.pallas import tpu as pltpu
from jax.experimental.pallas import tpu_sc as plsc

_N = 10000
_E = 320000
_NP = 10240
_EP = 327680
_PAD = _EP - _E
_DUMP = _N
_CHUNK = 128
_NTILES = 16
_RPT = _NP // _NTILES

@functools.cache
def _get_sc_deg(width=128):
    mesh = plsc.VectorSubcoreMesh(core_axis_name="c", subcore_axis_name="s")

    @functools.partial(
        pl.kernel,
        mesh=mesh,
        out_type=jax.ShapeDtypeStruct((2 * _NP, width), jnp.float32),
        scratch_types=[
            pltpu.VMEM((_EP // _CHUNK // 2 // _NTILES, 128), jnp.int32),
            pltpu.VMEM((_CHUNK, width), jnp.float32),
            pltpu.VMEM_SHARED((_NP, width), jnp.float32),
            pltpu.SemaphoreType.DMA((4,)),
        ],
    )
    def _sc_deg(dst_hbm, ones_hbm, zeros_hbm, out_hbm, dstall, onesbuf, acc,
                ssem):
        cid = lax.axis_index("c")
        tid = lax.axis_index("s")
        per_core = _EP // _CHUNK // 2
        per_tile = per_core // _NTILES
        base = cid * per_core + tid * per_tile
        pltpu.sync_copy(dst_hbm.at[pl.ds(base, per_tile)], dstall)
        pltpu.sync_copy(ones_hbm, onesbuf)
        for k in range(_RPT // _CHUNK):
            pltpu.sync_copy(zeros_hbm,
                            acc.at[pl.ds(tid * _RPT + k * _CHUNK, _CHUNK)])
        plsc.subcore_barrier()

        def body(i, carry):
            s = lax.rem(i, 4)

            @pl.when(i >= 4)
            def _wait():
                pltpu.make_async_copy(onesbuf, acc.at[dstall.at[i - 4]],
                                      ssem.at[s]).wait()

            pltpu.async_copy(onesbuf, acc.at[dstall.at[i]], ssem.at[s],
                             add=True)
            return carry

        lax.fori_loop(0, per_tile, body, 0)
        for k in range(4):
            s = (per_tile - 4 + k) % 4
            pltpu.make_async_copy(onesbuf, acc.at[dstall.at[per_tile - 4 + k]],
                                  ssem.at[s]).wait()
        plsc.subcore_barrier()
        pltpu.sync_copy(acc.at[pl.ds(tid * _RPT, _RPT)],
                        out_hbm.at[pl.ds(cid * _NP + tid * _RPT, _RPT)])

    return _sc_deg


@functools.cache
def _make_sc_pass(table_rows, per_core_chunks):
    del table_rows
    per_tile = per_core_chunks // _NTILES
    iblk = 40
    nblocks = per_tile // iblk
    mesh = plsc.VectorSubcoreMesh(core_axis_name="c", subcore_axis_name="s")

    @functools.partial(
        pl.kernel,
        mesh=mesh,
        out_type=jax.ShapeDtypeStruct((2 * _NP, 128), jnp.float32),
        scratch_types=[
            pltpu.VMEM((iblk, 128), jnp.int32),
            pltpu.VMEM((iblk, 128), jnp.int32),
            pltpu.VMEM((2, _CHUNK, 128), jnp.float32),
            pltpu.VMEM_SHARED((_NP, 128), jnp.float32),
            pltpu.SemaphoreType.DMA((2,)),
            pltpu.SemaphoreType.DMA((2,)),
        ],
    )
    def _sc_pass(table_hbm, src_hbm, dst_hbm, zeros_hbm, out_hbm,
                 srcall, dstall, gbuf, acc, gsem, ssem):
        cid = lax.axis_index("c")
        tid = lax.axis_index("s")
        base = cid * per_core_chunks + tid * per_tile
        pltpu.sync_copy(zeros_hbm, gbuf.at[0])
        for k in range(_RPT // _CHUNK):
            pltpu.sync_copy(gbuf.at[0],
                            acc.at[pl.ds(tid * _RPT + k * _CHUNK, _CHUNK)])
        plsc.subcore_barrier()

        def outer(bi, carry):
            blk = base + bi * iblk
            pltpu.sync_copy(src_hbm.at[pl.ds(blk, iblk)], srcall)
            pltpu.sync_copy(dst_hbm.at[pl.ds(blk, iblk)], dstall)
            pltpu.async_copy(table_hbm.at[srcall.at[0]], gbuf.at[0], gsem.at[0])
            for i in range(iblk):
                b = i % 2
                nb = 1 - b
                if i + 1 < iblk:
                    if i >= 1:
                        pltpu.make_async_copy(gbuf.at[nb],
                                              acc.at[dstall.at[i - 1]],
                                              ssem.at[nb]).wait()
                    pltpu.async_copy(table_hbm.at[srcall.at[i + 1]],
                                     gbuf.at[nb], gsem.at[nb])
                pltpu.make_async_copy(table_hbm.at[srcall.at[i]], gbuf.at[b],
                                      gsem.at[b]).wait()
                pltpu.async_copy(gbuf.at[b], acc.at[dstall.at[i]], ssem.at[b],
                                 add=True)
            for i in (iblk - 2, iblk - 1):
                pltpu.make_async_copy(gbuf.at[i % 2], acc.at[dstall.at[i]],
                                      ssem.at[i % 2]).wait()
            return carry

        lax.fori_loop(0, nblocks, outer, 0)
        plsc.subcore_barrier()
        pltpu.sync_copy(acc.at[pl.ds(tid * _RPT, _RPT)],
                        out_hbm.at[pl.ds(cid * _NP + tid * _RPT, _RPT)])

    return _sc_pass


_ROWS = 512
_NB = _NP // _ROWS

_DN = (((1,), (0,)), ((), ()))


def _dinv_of(d0, d1):
    return lax.rsqrt(d0[:, 0:1] + d1[:, 0:1] + 1.0)


def _tc1_body(x_ref, w1_ref, d0_ref, d1_ref, o_ref):
    dinv = _dinv_of(d0_ref, d1_ref)
    acc = lax.dot_general(x_ref[...], w1_ref[...], _DN,
                          precision=lax.Precision.DEFAULT,
                          preferred_element_type=jnp.float32)
    o_ref[...] = (acc * dinv)[None]


_tc1 = pl.pallas_call(
    _tc1_body,
    grid=(2, _NB := _NP // 512),
    in_specs=[
        pl.BlockSpec((512, 128), lambda h, i: (i, 0)),
        pl.BlockSpec((128, 128), lambda h, i: (0, h)),
        pl.BlockSpec((512, 128), lambda h, i: (i, 0)),
        pl.BlockSpec((512, 128), lambda h, i: (i, 0)),
    ],
    out_specs=pl.BlockSpec((1, 512, 128), lambda h, i: (h, i, 0)),
    out_shape=jax.ShapeDtypeStruct((2, _NP, 128), jnp.float32),
)


def _tc2_body(a0_ref, a1_ref, p0_ref, p1_ref, d0_ref, d1_ref, b1_ref, w2_ref,
              o_ref):
    dinv = _dinv_of(d0_ref, d1_ref)
    h_a = jnp.maximum(dinv * (a0_ref[...] + p0_ref[...]) + b1_ref[0:1, :], 0.0)
    h_b = jnp.maximum(dinv * (a1_ref[...] + p1_ref[...]) + b1_ref[1:2, :], 0.0)
    acc = lax.dot_general(h_a, w2_ref[0:128, :], _DN,
                          precision=lax.Precision.DEFAULT,
                          preferred_element_type=jnp.float32)
    acc += lax.dot_general(h_b, w2_ref[128:256, :], _DN,
                           precision=lax.Precision.DEFAULT,
                           preferred_element_type=jnp.float32)
    o_ref[...] = acc * dinv


_tc2 = pl.pallas_call(
    _tc2_body,
    grid=(_NB,),
    in_specs=[
        pl.BlockSpec((512, 128), lambda i: (i, 0)),
        pl.BlockSpec((512, 128), lambda i: (i, 0)),
        pl.BlockSpec((512, 128), lambda i: (i, 0)),
        pl.BlockSpec((512, 128), lambda i: (i, 0)),
        pl.BlockSpec((512, 128), lambda i: (i, 0)),
        pl.BlockSpec((512, 128), lambda i: (i, 0)),
        pl.BlockSpec((2, 128), lambda i: (0, 0)),
        pl.BlockSpec((256, 128), lambda i: (0, 0)),
    ],
    out_specs=pl.BlockSpec((512, 128), lambda i: (i, 0)),
    out_shape=jax.ShapeDtypeStruct((_NP, 128), jnp.float32),
)


def _tc3_body(a0_ref, a1_ref, p2_ref, d0_ref, d1_ref, b2_ref, o_ref):
    dinv = _dinv_of(d0_ref, d1_ref)
    o_ref[...] = dinv * (a0_ref[...] + a1_ref[...] + p2_ref[...]) + b2_ref[...]


_tc3 = pl.pallas_call(
    _tc3_body,
    grid=(_NB,),
    in_specs=[
        pl.BlockSpec((512, 128), lambda i: (i, 0)),
        pl.BlockSpec((512, 128), lambda i: (i, 0)),
        pl.BlockSpec((512, 128), lambda i: (i, 0)),
        pl.BlockSpec((512, 128), lambda i: (i, 0)),
        pl.BlockSpec((512, 128), lambda i: (i, 0)),
        pl.BlockSpec((1, 128), lambda i: (0, 0)),
    ],
    out_specs=pl.BlockSpec((512, 128), lambda i: (i, 0)),
    out_shape=jax.ShapeDtypeStruct((_NP, 128), jnp.float32),
)


def kernel(x, edge_index, w1, b1, w2, b2):
    f32 = jnp.float32
    src = edge_index[0].astype(jnp.int32)
    dst = edge_index[1].astype(jnp.int32)
    pad_idx = jnp.arange(_PAD, dtype=jnp.int32)
    src_p = jnp.concatenate([src, pad_idx % _N])
    dst_p = jnp.concatenate([dst, _DUMP + pad_idx % (_NP - _N)])
    src2 = jnp.concatenate([src_p, src_p + _NP])
    dst2 = jnp.concatenate([dst_p, dst_p])
    src_p = src_p.reshape(-1, _CHUNK)
    dst_p = dst_p.reshape(-1, _CHUNK)
    src2 = src2.reshape(-1, _CHUNK)
    dst2 = dst2.reshape(-1, _CHUNK)
    zeros128 = jnp.zeros((_CHUNK, 128), f32)
    ones128 = jnp.ones((_CHUNK, 128), f32)
    xp = jnp.pad(x, ((0, _NP - _N), (0, 0)))

    degpart = _get_sc_deg()(dst_p, ones128, zeros128)
    d0, d1 = degpart[:_NP], degpart[_NP:]
    p1s = _tc1(xp, w1, d0, d1)
    p1flat = p1s.reshape(2 * _NP, 128)
    a1 = _make_sc_pass(2 * _NP, _EP // _CHUNK)(p1flat, src2, dst2, zeros128)
    p2 = _tc2(a1[:_NP], a1[_NP:], p1s[0], p1s[1], d0, d1,
              b1.reshape(2, 128), w2)
    a2 = _make_sc_pass(_NP, _EP // _CHUNK // 2)(p2, src_p, dst_p, zeros128)
    out = _tc3(a2[:_NP], a2[_NP:], p2, d0, d1, b2.reshape(1, 128))
    return out[:_N]


# speedup vs baseline: 1.0251x; 1.0232x over previous
import functools

import jax
import jax.numpy as jnp
from jax import lax
from jax.experimental import pallas as pl
from jax.experimental.pallas import tpu as pltpu
from jax.experimental.pallas import tpu_sc as plsc

_N = 10000
_E = 320000
_NP = 10240
_EP = 327680
_PAD = _EP - _E
_DUMP = _N
_CHUNK = 128
_NTILES = 16
_RPT = _NP // _NTILES

@functools.cache
def _get_sc_deg(width=128):
    mesh = plsc.VectorSubcoreMesh(core_axis_name="c", subcore_axis_name="s")

    @functools.partial(
        pl.kernel,
        mesh=mesh,
        out_type=jax.ShapeDtypeStruct((2 * _NP, width), jnp.float32),
        scratch_types=[
            pltpu.VMEM((_EP // _CHUNK // 2 // _NTILES, 128), jnp.int32),
            pltpu.VMEM((_CHUNK, width), jnp.float32),
            pltpu.VMEM_SHARED((_NP, width), jnp.float32),
            pltpu.SemaphoreType.DMA((4,)),
        ],
    )
    def _sc_deg(dst_hbm, ones_hbm, zeros_hbm, out_hbm, dstall, onesbuf, acc,
                ssem):
        cid = lax.axis_index("c")
        tid = lax.axis_index("s")
        per_core = _EP // _CHUNK // 2
        per_tile = per_core // _NTILES
        base = cid * per_core + tid * per_tile
        pltpu.sync_copy(dst_hbm.at[pl.ds(base, per_tile)], dstall)
        pltpu.sync_copy(ones_hbm, onesbuf)
        for k in range(_RPT // _CHUNK):
            pltpu.sync_copy(zeros_hbm,
                            acc.at[pl.ds(tid * _RPT + k * _CHUNK, _CHUNK)])
        plsc.subcore_barrier()

        def body(i, carry):
            s = lax.rem(i, 4)

            @pl.when(i >= 4)
            def _wait():
                pltpu.make_async_copy(onesbuf, acc.at[dstall.at[i - 4]],
                                      ssem.at[s]).wait()

            pltpu.async_copy(onesbuf, acc.at[dstall.at[i]], ssem.at[s],
                             add=True)
            return carry

        lax.fori_loop(0, per_tile, body, 0)
        for k in range(4):
            s = (per_tile - 4 + k) % 4
            pltpu.make_async_copy(onesbuf, acc.at[dstall.at[per_tile - 4 + k]],
                                  ssem.at[s]).wait()
        plsc.subcore_barrier()
        pltpu.sync_copy(acc.at[pl.ds(tid * _RPT, _RPT)],
                        out_hbm.at[pl.ds(cid * _NP + tid * _RPT, _RPT)])

    return _sc_deg


@functools.cache
def _make_sc_pass(table_rows, per_core_chunks):
    del table_rows
    per_tile = per_core_chunks // _NTILES
    iblk = 40
    nblocks = per_tile // iblk
    mesh = plsc.VectorSubcoreMesh(core_axis_name="c", subcore_axis_name="s")

    @functools.partial(
        pl.kernel,
        mesh=mesh,
        out_type=jax.ShapeDtypeStruct((2 * _NP, 128), jnp.float32),
        scratch_types=[
            pltpu.VMEM((iblk, 128), jnp.int32),
            pltpu.VMEM((iblk, 128), jnp.int32),
            pltpu.VMEM((2, _CHUNK, 128), jnp.float32),
            pltpu.VMEM_SHARED((_NP, 128), jnp.float32),
            pltpu.SemaphoreType.DMA((2,)),
            pltpu.SemaphoreType.DMA((2,)),
        ],
    )
    def _sc_pass(table_hbm, src_hbm, dst_hbm, zeros_hbm, out_hbm,
                 srcall, dstall, gbuf, acc, gsem, ssem):
        cid = lax.axis_index("c")
        tid = lax.axis_index("s")
        base = cid * per_core_chunks + tid * per_tile
        pltpu.sync_copy(zeros_hbm, gbuf.at[0])
        for k in range(_RPT // _CHUNK):
            pltpu.sync_copy(gbuf.at[0],
                            acc.at[pl.ds(tid * _RPT + k * _CHUNK, _CHUNK)])
        plsc.subcore_barrier()

        def outer(bi, carry):
            blk = base + bi * iblk
            pltpu.sync_copy(src_hbm.at[pl.ds(blk, iblk)], srcall)
            pltpu.sync_copy(dst_hbm.at[pl.ds(blk, iblk)], dstall)
            pltpu.async_copy(table_hbm.at[srcall.at[0]], gbuf.at[0], gsem.at[0])
            for i in range(iblk):
                b = i % 2
                nb = 1 - b
                if i + 1 < iblk:
                    if i >= 1:
                        pltpu.make_async_copy(gbuf.at[nb],
                                              acc.at[dstall.at[i - 1]],
                                              ssem.at[nb]).wait()
                    pltpu.async_copy(table_hbm.at[srcall.at[i + 1]],
                                     gbuf.at[nb], gsem.at[nb])
                pltpu.make_async_copy(table_hbm.at[srcall.at[i]], gbuf.at[b],
                                      gsem.at[b]).wait()
                pltpu.async_copy(gbuf.at[b], acc.at[dstall.at[i]], ssem.at[b],
                                 add=True)
            for i in (iblk - 2, iblk - 1):
                pltpu.make_async_copy(gbuf.at[i % 2], acc.at[dstall.at[i]],
                                      ssem.at[i % 2]).wait()
            return carry

        lax.fori_loop(0, nblocks, outer, 0)
        plsc.subcore_barrier()
        pltpu.sync_copy(acc.at[pl.ds(tid * _RPT, _RPT)],
                        out_hbm.at[pl.ds(cid * _NP + tid * _RPT, _RPT)])

    return _sc_pass


_ROWS = 512
_NB = _NP // _ROWS

_DN = (((1,), (0,)), ((), ()))


def _dinv_of(d0, d1):
    return lax.rsqrt(d0[:, 0:1] + d1[:, 0:1] + 1.0)


def _tc1a_body(x_ref, w1_ref, o_ref):
    acc = lax.dot_general(x_ref[...], w1_ref[...], _DN,
                          precision=lax.Precision.DEFAULT,
                          preferred_element_type=jnp.float32)
    o_ref[...] = acc[None]


_tc1a = pl.pallas_call(
    _tc1a_body,
    grid=(2, _NB := _NP // 512),
    in_specs=[
        pl.BlockSpec((512, 128), lambda h, i: (i, 0)),
        pl.BlockSpec((128, 128), lambda h, i: (0, h)),
    ],
    out_specs=pl.BlockSpec((1, 512, 128), lambda h, i: (h, i, 0)),
    out_shape=jax.ShapeDtypeStruct((2, _NP, 128), jnp.float32),
)


def _tc1b_body(q0_ref, q1_ref, d0_ref, d1_ref, o_ref):
    dinv = _dinv_of(d0_ref, d1_ref)
    o_ref[...] = jnp.stack([q0_ref[...] * dinv, q1_ref[...] * dinv])


_tc1b = pl.pallas_call(
    _tc1b_body,
    grid=(_NB,),
    in_specs=[
        pl.BlockSpec((512, 128), lambda i: (i, 0)),
        pl.BlockSpec((512, 128), lambda i: (i, 0)),
        pl.BlockSpec((512, 128), lambda i: (i, 0)),
        pl.BlockSpec((512, 128), lambda i: (i, 0)),
    ],
    out_specs=pl.BlockSpec((2, 512, 128), lambda i: (0, i, 0)),
    out_shape=jax.ShapeDtypeStruct((2, _NP, 128), jnp.float32),
)


def _tc2_body(a0_ref, a1_ref, p0_ref, p1_ref, d0_ref, d1_ref, b1_ref, w2_ref,
              o_ref):
    dinv = _dinv_of(d0_ref, d1_ref)
    h_a = jnp.maximum(dinv * (a0_ref[...] + p0_ref[...]) + b1_ref[0:1, :], 0.0)
    h_b = jnp.maximum(dinv * (a1_ref[...] + p1_ref[...]) + b1_ref[1:2, :], 0.0)
    acc = lax.dot_general(h_a, w2_ref[0:128, :], _DN,
                          precision=lax.Precision.DEFAULT,
                          preferred_element_type=jnp.float32)
    acc += lax.dot_general(h_b, w2_ref[128:256, :], _DN,
                           precision=lax.Precision.DEFAULT,
                           preferred_element_type=jnp.float32)
    o_ref[...] = acc * dinv


_tc2 = pl.pallas_call(
    _tc2_body,
    grid=(_NB,),
    in_specs=[
        pl.BlockSpec((512, 128), lambda i: (i, 0)),
        pl.BlockSpec((512, 128), lambda i: (i, 0)),
        pl.BlockSpec((512, 128), lambda i: (i, 0)),
        pl.BlockSpec((512, 128), lambda i: (i, 0)),
        pl.BlockSpec((512, 128), lambda i: (i, 0)),
        pl.BlockSpec((512, 128), lambda i: (i, 0)),
        pl.BlockSpec((2, 128), lambda i: (0, 0)),
        pl.BlockSpec((256, 128), lambda i: (0, 0)),
    ],
    out_specs=pl.BlockSpec((512, 128), lambda i: (i, 0)),
    out_shape=jax.ShapeDtypeStruct((_NP, 128), jnp.float32),
)


def _tc3_body(a0_ref, a1_ref, p2_ref, d0_ref, d1_ref, b2_ref, o_ref):
    dinv = _dinv_of(d0_ref, d1_ref)
    o_ref[...] = dinv * (a0_ref[...] + a1_ref[...] + p2_ref[...]) + b2_ref[...]


_tc3 = pl.pallas_call(
    _tc3_body,
    grid=(_NB,),
    in_specs=[
        pl.BlockSpec((512, 128), lambda i: (i, 0)),
        pl.BlockSpec((512, 128), lambda i: (i, 0)),
        pl.BlockSpec((512, 128), lambda i: (i, 0)),
        pl.BlockSpec((512, 128), lambda i: (i, 0)),
        pl.BlockSpec((512, 128), lambda i: (i, 0)),
        pl.BlockSpec((1, 128), lambda i: (0, 0)),
    ],
    out_specs=pl.BlockSpec((512, 128), lambda i: (i, 0)),
    out_shape=jax.ShapeDtypeStruct((_NP, 128), jnp.float32),
)


def kernel(x, edge_index, w1, b1, w2, b2):
    f32 = jnp.float32
    src = edge_index[0].astype(jnp.int32)
    dst = edge_index[1].astype(jnp.int32)
    pad_idx = jnp.arange(_PAD, dtype=jnp.int32)
    src_p = jnp.concatenate([src, pad_idx % _N])
    dst_p = jnp.concatenate([dst, _DUMP + pad_idx % (_NP - _N)])
    src2 = jnp.concatenate([src_p, src_p + _NP])
    dst2 = jnp.concatenate([dst_p, dst_p])
    src_p = src_p.reshape(-1, _CHUNK)
    dst_p = dst_p.reshape(-1, _CHUNK)
    src2 = src2.reshape(-1, _CHUNK)
    dst2 = dst2.reshape(-1, _CHUNK)
    zeros128 = jnp.zeros((_CHUNK, 128), f32)
    ones128 = jnp.ones((_CHUNK, 128), f32)
    xp = jnp.pad(x, ((0, _NP - _N), (0, 0)))

    q1s = _tc1a(xp, w1)
    degpart = _get_sc_deg()(dst_p, ones128, zeros128)
    d0, d1 = degpart[:_NP], degpart[_NP:]
    p1s = _tc1b(q1s[0], q1s[1], d0, d1)
    p1flat = p1s.reshape(2 * _NP, 128)
    a1 = _make_sc_pass(2 * _NP, _EP // _CHUNK)(p1flat, src2, dst2, zeros128)
    p2 = _tc2(a1[:_NP], a1[_NP:], p1s[0], p1s[1], d0, d1,
              b1.reshape(2, 128), w2)
    a2 = _make_sc_pass(_NP, _EP // _CHUNK // 2)(p2, src_p, dst_p, zeros128)
    out = _tc3(a2[:_NP], a2[_NP:], p2, d0, d1, b2.reshape(1, 128))
    return out[:_N]


# speedup vs baseline: 1.0793x; 1.0529x over previous
import functools

import jax
import jax.numpy as jnp
from jax import lax
from jax.experimental import pallas as pl
from jax.experimental.pallas import tpu as pltpu
from jax.experimental.pallas import tpu_sc as plsc

_N = 10000
_E = 320000
_NP = 10240
_EP = 327680
_PAD = _EP - _E
_DUMP = _N
_CHUNK = 128
_NTILES = 16
_RPT = _NP // _NTILES

@functools.cache
def _get_sc_deg(width=128):
    mesh = plsc.VectorSubcoreMesh(core_axis_name="c", subcore_axis_name="s")

    @functools.partial(
        pl.kernel,
        mesh=mesh,
        out_type=jax.ShapeDtypeStruct((2 * _NP, width), jnp.float32),
        scratch_types=[
            pltpu.VMEM((_EP // _CHUNK // 2 // _NTILES, 128), jnp.int32),
            pltpu.VMEM((_CHUNK, width), jnp.float32),
            pltpu.VMEM_SHARED((_NP, width), jnp.float32),
            pltpu.SemaphoreType.DMA((4,)),
        ],
    )
    def _sc_deg(dst_hbm, ones_hbm, zeros_hbm, out_hbm, dstall, onesbuf, acc,
                ssem):
        cid = lax.axis_index("c")
        tid = lax.axis_index("s")
        per_core = _EP // _CHUNK // 2
        per_tile = per_core // _NTILES
        base = cid * per_core + tid * per_tile
        pltpu.sync_copy(dst_hbm.at[pl.ds(base, per_tile)], dstall)
        pltpu.sync_copy(ones_hbm, onesbuf)
        for k in range(_RPT // _CHUNK):
            pltpu.sync_copy(zeros_hbm,
                            acc.at[pl.ds(tid * _RPT + k * _CHUNK, _CHUNK)])
        plsc.subcore_barrier()

        def body(i, carry):
            s = lax.rem(i, 4)

            @pl.when(i >= 4)
            def _wait():
                pltpu.make_async_copy(onesbuf, acc.at[dstall.at[i - 4]],
                                      ssem.at[s]).wait()

            pltpu.async_copy(onesbuf, acc.at[dstall.at[i]], ssem.at[s],
                             add=True)
            return carry

        lax.fori_loop(0, per_tile, body, 0)
        for k in range(4):
            s = (per_tile - 4 + k) % 4
            pltpu.make_async_copy(onesbuf, acc.at[dstall.at[per_tile - 4 + k]],
                                  ssem.at[s]).wait()
        plsc.subcore_barrier()
        pltpu.sync_copy(acc.at[pl.ds(tid * _RPT, _RPT)],
                        out_hbm.at[pl.ds(cid * _NP + tid * _RPT, _RPT)])

    return _sc_deg


@functools.cache
def _make_sc_pass(table_rows, per_core_chunks):
    del table_rows
    per_tile = per_core_chunks // _NTILES
    iblk = 40
    nblocks = per_tile // iblk
    mesh = plsc.VectorSubcoreMesh(core_axis_name="c", subcore_axis_name="s")

    @functools.partial(
        pl.kernel,
        mesh=mesh,
        out_type=jax.ShapeDtypeStruct((2 * _NP, 128), jnp.float32),
        scratch_types=[
            pltpu.VMEM((iblk, 128), jnp.int32),
            pltpu.VMEM((iblk, 128), jnp.int32),
            pltpu.VMEM((2, _CHUNK, 128), jnp.float32),
            pltpu.VMEM_SHARED((_NP, 128), jnp.float32),
            pltpu.SemaphoreType.DMA((2,)),
            pltpu.SemaphoreType.DMA((2,)),
        ],
    )
    def _sc_pass(table_hbm, src_hbm, dst_hbm, zeros_hbm, out_hbm,
                 srcall, dstall, gbuf, acc, gsem, ssem):
        cid = lax.axis_index("c")
        tid = lax.axis_index("s")
        base = cid * per_core_chunks + tid * per_tile
        pltpu.sync_copy(zeros_hbm, gbuf.at[0])
        for k in range(_RPT // _CHUNK):
            pltpu.sync_copy(gbuf.at[0],
                            acc.at[pl.ds(tid * _RPT + k * _CHUNK, _CHUNK)])
        plsc.subcore_barrier()

        def outer(bi, carry):
            blk = base + bi * iblk
            pltpu.sync_copy(src_hbm.at[pl.ds(blk, iblk)], srcall)
            pltpu.sync_copy(dst_hbm.at[pl.ds(blk, iblk)], dstall)
            pltpu.async_copy(table_hbm.at[srcall.at[0]], gbuf.at[0], gsem.at[0])
            for i in range(iblk):
                b = i % 2
                nb = 1 - b
                if i + 1 < iblk:
                    if i >= 1:
                        pltpu.make_async_copy(gbuf.at[nb],
                                              acc.at[dstall.at[i - 1]],
                                              ssem.at[nb]).wait()
                    pltpu.async_copy(table_hbm.at[srcall.at[i + 1]],
                                     gbuf.at[nb], gsem.at[nb])
                pltpu.make_async_copy(table_hbm.at[srcall.at[i]], gbuf.at[b],
                                      gsem.at[b]).wait()
                pltpu.async_copy(gbuf.at[b], acc.at[dstall.at[i]], ssem.at[b],
                                 add=True)
            for i in (iblk - 2, iblk - 1):
                pltpu.make_async_copy(gbuf.at[i % 2], acc.at[dstall.at[i]],
                                      ssem.at[i % 2]).wait()
            return carry

        lax.fori_loop(0, nblocks, outer, 0)
        plsc.subcore_barrier()
        pltpu.sync_copy(acc.at[pl.ds(tid * _RPT, _RPT)],
                        out_hbm.at[pl.ds(cid * _NP + tid * _RPT, _RPT)])

    return _sc_pass


_ROWS = 512
_NB = _NP // _ROWS

_DN = (((1,), (0,)), ((), ()))


def _dinv_of(d0, d1):
    return lax.rsqrt(d0[:, 0:1] + d1[:, 0:1] + 1.0)


def _tc1a_body(x_ref, w1_ref, o_ref):
    acc = lax.dot_general(x_ref[...], w1_ref[...], _DN,
                          precision=lax.Precision.DEFAULT,
                          preferred_element_type=jnp.float32)
    o_ref[...] = acc[None]


_tc1a = pl.pallas_call(
    _tc1a_body,
    grid=(2, _NB := _NP // 512),
    in_specs=[
        pl.BlockSpec((512, 128), lambda h, i: (i, 0)),
        pl.BlockSpec((128, 128), lambda h, i: (0, h)),
    ],
    out_specs=pl.BlockSpec((1, 512, 128), lambda h, i: (h, i, 0)),
    out_shape=jax.ShapeDtypeStruct((2, _NP, 128), jnp.float32),
)


def _tc1b_body(q0_ref, q1_ref, d0_ref, d1_ref, o_ref):
    dinv = _dinv_of(d0_ref, d1_ref)
    o_ref[...] = jnp.stack([q0_ref[0] * dinv, q1_ref[0] * dinv])


_tc1b = pl.pallas_call(
    _tc1b_body,
    grid=(_NB,),
    in_specs=[
        pl.BlockSpec((1, 512, 128), lambda i: (0, i, 0)),
        pl.BlockSpec((1, 512, 128), lambda i: (1, i, 0)),
        pl.BlockSpec((512, 128), lambda i: (i, 0)),
        pl.BlockSpec((512, 128), lambda i: (i + _NB, 0)),
    ],
    out_specs=pl.BlockSpec((2, 512, 128), lambda i: (0, i, 0)),
    out_shape=jax.ShapeDtypeStruct((2, _NP, 128), jnp.float32),
)


def _tc2_body(a0_ref, a1_ref, p0_ref, p1_ref, d0_ref, d1_ref, b1_ref, w2_ref,
              o_ref):
    dinv = _dinv_of(d0_ref, d1_ref)
    h_a = jnp.maximum(dinv * (a0_ref[...] + p0_ref[0]) + b1_ref[0:1, :], 0.0)
    h_b = jnp.maximum(dinv * (a1_ref[...] + p1_ref[0]) + b1_ref[1:2, :], 0.0)
    acc = lax.dot_general(h_a, w2_ref[0:128, :], _DN,
                          precision=lax.Precision.DEFAULT,
                          preferred_element_type=jnp.float32)
    acc += lax.dot_general(h_b, w2_ref[128:256, :], _DN,
                           precision=lax.Precision.DEFAULT,
                           preferred_element_type=jnp.float32)
    o_ref[...] = acc * dinv


_tc2 = pl.pallas_call(
    _tc2_body,
    grid=(_NB,),
    in_specs=[
        pl.BlockSpec((512, 128), lambda i: (i, 0)),
        pl.BlockSpec((512, 128), lambda i: (i + _NB, 0)),
        pl.BlockSpec((1, 512, 128), lambda i: (0, i, 0)),
        pl.BlockSpec((1, 512, 128), lambda i: (1, i, 0)),
        pl.BlockSpec((512, 128), lambda i: (i, 0)),
        pl.BlockSpec((512, 128), lambda i: (i + _NB, 0)),
        pl.BlockSpec((2, 128), lambda i: (0, 0)),
        pl.BlockSpec((256, 128), lambda i: (0, 0)),
    ],
    out_specs=pl.BlockSpec((512, 128), lambda i: (i, 0)),
    out_shape=jax.ShapeDtypeStruct((_NP, 128), jnp.float32),
)


def _tc3_body(a0_ref, a1_ref, p2_ref, d0_ref, d1_ref, b2_ref, o_ref):
    dinv = _dinv_of(d0_ref, d1_ref)
    o_ref[...] = dinv * (a0_ref[...] + a1_ref[...] + p2_ref[...]) + b2_ref[...]


_tc3 = pl.pallas_call(
    _tc3_body,
    grid=(_NB,),
    in_specs=[
        pl.BlockSpec((512, 128), lambda i: (i, 0)),
        pl.BlockSpec((512, 128), lambda i: (i + _NB, 0)),
        pl.BlockSpec((512, 128), lambda i: (i, 0)),
        pl.BlockSpec((512, 128), lambda i: (i, 0)),
        pl.BlockSpec((512, 128), lambda i: (i + _NB, 0)),
        pl.BlockSpec((1, 128), lambda i: (0, 0)),
    ],
    out_specs=pl.BlockSpec((512, 128), lambda i: (i, 0)),
    out_shape=jax.ShapeDtypeStruct((_NP, 128), jnp.float32),
)


def kernel(x, edge_index, w1, b1, w2, b2):
    f32 = jnp.float32
    src = edge_index[0].astype(jnp.int32)
    dst = edge_index[1].astype(jnp.int32)
    pad_idx = jnp.arange(_PAD, dtype=jnp.int32)
    src_p = jnp.concatenate([src, pad_idx % _N])
    dst_p = jnp.concatenate([dst, _DUMP + pad_idx % (_NP - _N)])
    src2 = jnp.concatenate([src_p, src_p + _NP])
    dst2 = jnp.concatenate([dst_p, dst_p])
    src_p = src_p.reshape(-1, _CHUNK)
    dst_p = dst_p.reshape(-1, _CHUNK)
    src2 = src2.reshape(-1, _CHUNK)
    dst2 = dst2.reshape(-1, _CHUNK)
    zeros128 = jnp.zeros((_CHUNK, 128), f32)
    ones128 = jnp.ones((_CHUNK, 128), f32)
    xp = jnp.pad(x, ((0, _NP - _N), (0, 0)))

    q1s = _tc1a(xp, w1)
    degpart = _get_sc_deg()(dst_p, ones128, zeros128)
    p1s = _tc1b(q1s, q1s, degpart, degpart)
    p1flat = p1s.reshape(2 * _NP, 128)
    a1 = _make_sc_pass(2 * _NP, _EP // _CHUNK)(p1flat, src2, dst2, zeros128)
    p2 = _tc2(a1, a1, p1s, p1s, degpart, degpart,
              b1.reshape(2, 128), w2)
    a2 = _make_sc_pass(_NP, _EP // _CHUNK // 2)(p2, src_p, dst_p, zeros128)
    out = _tc3(a2, a2, p2, degpart, degpart, b2.reshape(1, 128))
    return out[:_N]
